# baseline (speedup 1.0000x reference)
import functools

import jax
import jax.numpy as jnp
from jax import lax
from jax.experimental import pallas as pl
from jax.experimental.pallas import tpu as pltpu
from jax.experimental.pallas import tpu_sc as plsc

NUM_ROWS = 16384
NUM_COLS = 20
EMBED_DIM = 32
B = NUM_ROWS * NUM_COLS

NC = 2
NS = 16
NW = NC * NS
ROWS_PER_W = NUM_ROWS // NW
B_PER_W = B // NW
RBLK = 128
HALF = RBLK // 2
CHUNK = HALF * NUM_COLS
N_RBLK = ROWS_PER_W // RBLK

OUT5 = (NUM_COLS, EMBED_DIM // 8, NUM_ROWS // 128, 8, 128)


@functools.partial(
    pl.kernel,
    mesh=plsc.VectorSubcoreMesh(core_axis_name="c", subcore_axis_name="s"),
    out_type=(
        jax.ShapeDtypeStruct(OUT5, jnp.float32),
        jax.ShapeDtypeStruct(OUT5, jnp.float32),
    ),
    scratch_types=[
        pltpu.VMEM((CHUNK,), jnp.int32),
        pltpu.VMEM((CHUNK, EMBED_DIM), jnp.float32),
        pltpu.VMEM((NUM_COLS, EMBED_DIM // 8, 8, 128), jnp.float32),
        pltpu.SemaphoreType.DMA((2,)),
        pltpu.SemaphoreType.DMA,
    ],
    compiler_params=pltpu.CompilerParams(
        use_tc_tiling_on_sc=False, needs_layout_passes=False),
)
def _double_gather(w_sr, w_tg, idx_sr, idx_tg, o_sr, o_tg,
                   idx_v, rows_v, out_v, isem, gsem):
    wid = lax.axis_index("s") * NC + lax.axis_index("c")
    base = wid * B_PER_W
    r0 = wid * N_RBLK
    iota16 = lax.iota(jnp.int32, 16)
    iota_r = iota16 * NUM_COLS

    for t, (w, idx, o) in enumerate(((w_sr, idx_sr, o_sr), (w_tg, idx_tg, o_tg))):

        def rblk_body(rb, _, w=w, idx=idx, o=o):
            for half in range(2):
                pltpu.async_copy(
                    idx.at[pl.ds(base + (2 * rb + half) * CHUNK, CHUNK)],
                    idx_v, isem.at[0]).wait()
                pltpu.async_copy(w.at[idx_v], rows_v, gsem).wait()

                def fill_body(col, _, half=half):
                    for h in range(EMBED_DIM // 8):
                        for l in range(8):
                            cidx = jnp.full((16,), h * 8 + l, jnp.int32)
                            for k in range(0, HALF, 16):
                                ridx = iota_r + (k * NUM_COLS + col)
                                vals = plsc.load_gather(rows_v, [ridx, cidx])
                                out_v[col, h, l,
                                      pl.ds(half * HALF + k, 16)] = vals
                    return 0

                lax.fori_loop(0, NUM_COLS, fill_body, 0)

            def wb_body(col, _, o=o, rb=rb):
                for h in range(EMBED_DIM // 8):
                    pltpu.sync_copy(out_v.at[col, h], o.at[col, h, r0 + rb])
                return 0

            lax.fori_loop(0, NUM_COLS, wb_body, 0)
            return 0

        lax.fori_loop(0, N_RBLK, rblk_body, 0)


def kernel(sr_data, tg_data, W_sr, W_tg):
    idx_sr = sr_data.reshape(B)
    idx_tg = tg_data.reshape(B)
    o_sr, o_tg = _double_gather(W_sr, W_tg, idx_sr, idx_tg)

    def unpack(o5):
        return o5.transpose(2, 4, 0, 1, 3).reshape(NUM_ROWS, NUM_COLS, EMBED_DIM)

    return (unpack(o_sr), unpack(o_tg))


# speedup vs baseline: 1.2027x; 1.2027x over previous
import functools

import jax
import jax.numpy as jnp
from jax import lax
from jax.experimental import pallas as pl
from jax.experimental.pallas import tpu as pltpu
from jax.experimental.pallas import tpu_sc as plsc

NUM_ROWS = 16384
NUM_COLS = 20
EMBED_DIM = 32
B = NUM_ROWS * NUM_COLS

NC = 2
NS = 16
NW = NC * NS
ROWS_PER_W = NUM_ROWS // NW
B_PER_W = B // NW
RBLK = 128
HALF = RBLK // 2
CHUNK = HALF * NUM_COLS
N_RBLK = ROWS_PER_W // RBLK

OUT5 = (NUM_COLS, EMBED_DIM // 8, NUM_ROWS // 128, 8, 128)


@functools.partial(
    pl.kernel,
    mesh=plsc.VectorSubcoreMesh(core_axis_name="c", subcore_axis_name="s"),
    out_type=(
        jax.ShapeDtypeStruct(OUT5, jnp.float32),
        jax.ShapeDtypeStruct(OUT5, jnp.float32),
    ),
    scratch_types=[
        pltpu.VMEM((CHUNK,), jnp.int32),
        pltpu.VMEM((CHUNK, EMBED_DIM), jnp.float32),
        pltpu.VMEM((NUM_COLS, EMBED_DIM // 8, 8, 128), jnp.float32),
        pltpu.SemaphoreType.DMA((2,)),
        pltpu.SemaphoreType.DMA,
        pltpu.SemaphoreType.DMA,
    ],
    compiler_params=pltpu.CompilerParams(
        use_tc_tiling_on_sc=False, needs_layout_passes=False),
)
def _double_gather(w_sr, w_tg, idx_sr, idx_tg, o_sr, o_tg,
                   idx_v, rows_v, out_v, isem, gsem, wsem):
    wid = lax.axis_index("s") * NC + lax.axis_index("c")
    base = wid * B_PER_W
    r0 = wid * N_RBLK
    iota16 = lax.iota(jnp.int32, 16)
    iota_r = iota16 * NUM_COLS

    for t, (w, idx, o) in enumerate(((w_sr, idx_sr, o_sr), (w_tg, idx_tg, o_tg))):

        def rblk_body(rb, _, w=w, idx=idx, o=o):
            for half in range(2):
                pltpu.async_copy(
                    idx.at[pl.ds(base + (2 * rb + half) * CHUNK, CHUNK)],
                    idx_v, isem.at[0]).wait()
                pltpu.async_copy(w.at[idx_v], rows_v, gsem).wait()

                @plsc.parallel_loop(0, NUM_COLS, 1, unroll=2)
                def _fill(col, half=half):
                    for h in range(EMBED_DIM // 8):
                        for l in range(8):
                            cidx = jnp.full((16,), h * 8 + l, jnp.int32)
                            for k in range(0, HALF, 16):
                                ridx = iota_r + (k * NUM_COLS + col)
                                vals = plsc.load_gather(rows_v, [ridx, cidx])
                                out_v[col, h, l,
                                      pl.ds(half * HALF + k, 16)] = vals

            def wb_body(col, _, o=o, rb=rb):
                for h in range(EMBED_DIM // 8):
                    pltpu.async_copy(out_v.at[col, h], o.at[col, h, r0 + rb],
                                     wsem)
                return 0

            lax.fori_loop(0, NUM_COLS, wb_body, 0)

            def wb_drain(col, _, o=o, rb=rb):
                for h in range(EMBED_DIM // 8):
                    pltpu.make_async_copy(out_v.at[col, h],
                                          o.at[col, h, r0 + rb], wsem).wait()
                return 0

            lax.fori_loop(0, NUM_COLS, wb_drain, 0)
            return 0

        lax.fori_loop(0, N_RBLK, rblk_body, 0)


def kernel(sr_data, tg_data, W_sr, W_tg):
    idx_sr = sr_data.reshape(B)
    idx_tg = tg_data.reshape(B)
    o_sr, o_tg = _double_gather(W_sr, W_tg, idx_sr, idx_tg)

    def unpack(o5):
        return o5.transpose(2, 4, 0, 1, 3).reshape(NUM_ROWS, NUM_COLS, EMBED_DIM)

    return (unpack(o_sr), unpack(o_tg))
